# Initial kernel scaffold; baseline (speedup 1.0000x reference)
import jax
import jax.numpy as jnp
from jax.experimental import pallas as pl


def kernel(x, edge_index, W, b):
    raise NotImplementedError("write your pallas kernel here")


# speedup vs baseline: 256.4648x; 256.4648x over previous
import functools

import jax
import jax.numpy as jnp
from jax import lax
from jax.experimental import pallas as pl
from jax.experimental.pallas import tpu as pltpu
from jax.experimental.pallas import tpu_sc as plsc

ACC_SCALE = 0.02
MAX_VEL = 0.1

NC = 2
NS = 16


def _sc_degrees(src, dst, n):
    e = src.shape[0]
    epw = e // (NC * NS)
    ch = 8000
    assert epw % ch == 0 and epw * NC * NS == e
    mesh = plsc.VectorSubcoreMesh(core_axis_name="c", subcore_axis_name="s")

    @functools.partial(
        pl.kernel,
        out_type=(
            jax.ShapeDtypeStruct((NC, n), jnp.float32),
            jax.ShapeDtypeStruct((NC, n), jnp.float32),
        ),
        mesh=mesh,
        compiler_params=pltpu.CompilerParams(use_tc_tiling_on_sc=False),
        scratch_types=[
            pltpu.VMEM((ch,), jnp.int32),
            pltpu.VMEM((ch,), jnp.float32),
            pltpu.VMEM_SHARED((n,), jnp.float32),
            pltpu.VMEM_SHARED((n,), jnp.float32),
        ],
    )
    def deg_kernel(src_h, dst_h, zeros_h, ones_h, out_src, out_dst,
                   idx_v, ones_v, dsrc_sh, ddst_sh):
        c = lax.axis_index("c")
        s = lax.axis_index("s")

        @pl.when(s == 0)
        def _():
            pltpu.sync_copy(zeros_h, dsrc_sh)
            pltpu.sync_copy(zeros_h, ddst_sh)

        pltpu.sync_copy(ones_h, ones_v)
        plsc.subcore_barrier()

        base = (c * NS + s) * epw

        def body(i, carry):
            off = base + i * ch
            pltpu.sync_copy(src_h.at[pl.ds(off, ch)], idx_v)
            pltpu.sync_copy(ones_v, dsrc_sh.at[idx_v], add=True)
            pltpu.sync_copy(dst_h.at[pl.ds(off, ch)], idx_v)
            pltpu.sync_copy(ones_v, ddst_sh.at[idx_v], add=True)
            return carry

        lax.fori_loop(0, epw // ch, body, 0)
        plsc.subcore_barrier()

        @pl.when(s == 0)
        def _():
            pltpu.sync_copy(dsrc_sh, out_src.at[c])
            pltpu.sync_copy(ddst_sh, out_dst.at[c])

    zeros = jnp.zeros((n,), jnp.float32)
    ones = jnp.ones((ch,), jnp.float32)
    return deg_kernel(src, dst, zeros, ones)


def _sc_messages(src, dst, yt, n):
    e = src.shape[0]
    epw = e // (NC * NS)
    ch = 8000
    assert epw % ch == 0
    mesh = plsc.VectorSubcoreMesh(core_axis_name="c", subcore_axis_name="s")

    @functools.partial(
        pl.kernel,
        out_type=jax.ShapeDtypeStruct((NC, 2, n), jnp.float32),
        mesh=mesh,
        compiler_params=pltpu.CompilerParams(use_tc_tiling_on_sc=False),
        scratch_types=[
            pltpu.VMEM((ch,), jnp.int32),
            pltpu.VMEM((ch,), jnp.int32),
            pltpu.VMEM((ch,), jnp.float32),
            pltpu.VMEM((ch,), jnp.float32),
            pltpu.VMEM_SHARED((n,), jnp.float32),
            pltpu.VMEM_SHARED((n,), jnp.float32),
            pltpu.VMEM_SHARED((n,), jnp.float32),
            pltpu.VMEM_SHARED((n,), jnp.float32),
        ],
    )
    def msg_kernel(src_h, dst_h, y_h, zeros_h, out_acc,
                   idx_s, idx_d, v0, v1, y0_sh, y1_sh, a0_sh, a1_sh):
        c = lax.axis_index("c")
        s = lax.axis_index("s")

        @pl.when(s == 0)
        def _():
            pltpu.sync_copy(y_h.at[0], y0_sh)
            pltpu.sync_copy(y_h.at[1], y1_sh)
            pltpu.sync_copy(zeros_h, a0_sh)
            pltpu.sync_copy(zeros_h, a1_sh)

        plsc.subcore_barrier()

        base = (c * NS + s) * epw

        def body(i, carry):
            off = base + i * ch
            pltpu.sync_copy(src_h.at[pl.ds(off, ch)], idx_s)
            pltpu.sync_copy(dst_h.at[pl.ds(off, ch)], idx_d)
            pltpu.sync_copy(y0_sh.at[idx_s], v0)
            pltpu.sync_copy(y1_sh.at[idx_s], v1)
            pltpu.sync_copy(v0, a0_sh.at[idx_d], add=True)
            pltpu.sync_copy(v1, a1_sh.at[idx_d], add=True)
            return carry

        lax.fori_loop(0, epw // ch, body, 0)
        plsc.subcore_barrier()

        @pl.when(s == 0)
        def _():
            pltpu.sync_copy(a0_sh, out_acc.at[c, 0])
            pltpu.sync_copy(a1_sh, out_acc.at[c, 1])

    zeros = jnp.zeros((n,), jnp.float32)
    return msg_kernel(src, dst, yt, zeros)


def _tc_mid_body(xt_ref, w_ref, degp_ref, yt_ref, dinv_ref):
    deg = degp_ref[0:1, :] + degp_ref[1:2, :] + 1.0
    dinv = lax.rsqrt(deg)
    dinv_ref[...] = dinv
    for j in range(2):
        xw = xt_ref[0:1, :] * w_ref[0, j]
        for cc in range(1, 5):
            xw = xw + xt_ref[cc:cc + 1, :] * w_ref[cc, j]
        yt_ref[j:j + 1, :] = xw * dinv


def _tc_mid(xt, w, degp, n):
    return pl.pallas_call(
        _tc_mid_body,
        out_shape=(
            jax.ShapeDtypeStruct((2, n), jnp.float32),
            jax.ShapeDtypeStruct((1, n), jnp.float32),
        ),
        in_specs=[
            pl.BlockSpec(memory_space=pltpu.VMEM),
            pl.BlockSpec(memory_space=pltpu.SMEM),
            pl.BlockSpec(memory_space=pltpu.VMEM),
        ],
        out_specs=(
            pl.BlockSpec(memory_space=pltpu.VMEM),
            pl.BlockSpec(memory_space=pltpu.VMEM),
        ),
    )(xt, w, degp)


def _tc_final_body(xt_ref, yt_ref, dinv_ref, acct_ref, degsp_ref, b_ref,
                   newxt_ref, vb_ref, pp_ref, bc_ref, fr_ref):
    n = xt_ref.shape[1]
    dinv = dinv_ref[0:1, :]
    food_mask = (xt_ref[4:5, :] == 1.0).astype(jnp.float32)
    vb = []
    pp = []
    bc = 0.0
    for j in range(2):
        acc = acct_ref[0, j:j + 1, :] + acct_ref[1, j:j + 1, :]
        h = dinv * (yt_ref[j:j + 1, :] + acc) + b_ref[j]
        a = h * ACC_SCALE * food_mask
        vel = jnp.clip(xt_ref[2 + j:3 + j, :] + a, -MAX_VEL, MAX_VEL)
        pos = xt_ref[j:j + 1, :] + vel
        newxt_ref[j:j + 1, :] = pos
        newxt_ref[2 + j:3 + j, :] = vel
        apos = jnp.abs(pos)
        bc = bc + jnp.sum(jnp.where(apos > 1.0, jnp.log(apos), 0.0))
        vb.append(jnp.sum(jnp.abs(vel)) / n)
        pp.append(jnp.sum(apos) / n)
    newxt_ref[4:5, :] = xt_ref[4:5, :]
    deg_src = degsp_ref[0:1, :] + degsp_ref[1:2, :]
    fr = jnp.sum(jnp.where((xt_ref[4:5, :] == 0.0) & (deg_src > 4.0),
                           1.0, 0.0))
    vb_ref[0] = vb[0]
    vb_ref[1] = vb[1]
    pp_ref[0] = pp[0]
    pp_ref[1] = pp[1]
    bc_ref[0] = bc
    fr_ref[0] = fr


def _tc_final(xt, yt, dinv, acct, degsp, b, n):
    return pl.pallas_call(
        _tc_final_body,
        out_shape=(
            jax.ShapeDtypeStruct((5, n), jnp.float32),
            jax.ShapeDtypeStruct((2,), jnp.float32),
            jax.ShapeDtypeStruct((2,), jnp.float32),
            jax.ShapeDtypeStruct((1,), jnp.float32),
            jax.ShapeDtypeStruct((1,), jnp.float32),
        ),
        in_specs=[
            pl.BlockSpec(memory_space=pltpu.VMEM),
            pl.BlockSpec(memory_space=pltpu.VMEM),
            pl.BlockSpec(memory_space=pltpu.VMEM),
            pl.BlockSpec(memory_space=pltpu.VMEM),
            pl.BlockSpec(memory_space=pltpu.VMEM),
            pl.BlockSpec(memory_space=pltpu.SMEM),
        ],
        out_specs=(
            pl.BlockSpec(memory_space=pltpu.VMEM),
            pl.BlockSpec(memory_space=pltpu.SMEM),
            pl.BlockSpec(memory_space=pltpu.SMEM),
            pl.BlockSpec(memory_space=pltpu.SMEM),
            pl.BlockSpec(memory_space=pltpu.SMEM),
        ),
    )(xt, yt, dinv, acct, degsp, b)


def kernel(x, edge_index, W, b):
    n = x.shape[0]
    src = edge_index[0]
    dst = edge_index[1]

    degsp, degdp = _sc_degrees(src, dst, n)

    xt = x.T
    yt, dinv = _tc_mid(xt, W, degdp, n)

    acct = _sc_messages(src, dst, yt, n)

    newxt, vb, pp, bc, fr = _tc_final(xt, yt, dinv, acct, degsp, b, n)
    return (newxt.T, vb, pp, bc[0], fr[0])
